# baseline (speedup 1.0000x reference)
import functools

import jax
import jax.numpy as jnp
from jax.experimental import pallas as pl
from jax.experimental.pallas import tpu as pltpu

_CHUNK = 512


def _scorer_body(nq, hw, nb, b_neigh, fv_ref, mb_ref, pix_ref, img_ref,
                 acc_ref, d2_ref):
    batch = nq // hw
    n_chunks = mb_ref.shape[0] // _CHUNK
    fv = fv_ref[...]
    fvm2 = fv * (-2.0)
    ones_row = jnp.ones((1, fv.shape[1]), jnp.float32)

    acc_ref[...] = jnp.full(acc_ref.shape, jnp.inf, jnp.float32)

    def min_step(i, _):
        mbc = mb_ref[pl.ds(i * _CHUNK, _CHUNK), :]
        mbn = jax.lax.dot_general(ones_row, mbc * mbc,
                                  (((1,), (1,)), ((), ())),
                                  preferred_element_type=jnp.float32)
        prod = jax.lax.dot_general(fvm2, mbc,
                                   (((1,), (1,)), ((), ())),
                                   preferred_element_type=jnp.float32)
        cmin = jnp.min(prod + mbn, axis=1, keepdims=True)
        acc_ref[...] = jnp.minimum(acc_ref[...], cmin)
        return 0

    jax.lax.fori_loop(0, n_chunks, min_step, 0)

    fvn = jnp.sum(fv * fv, axis=1, keepdims=True)
    pix = jnp.sqrt(acc_ref[...] + fvn)
    pix_ref[...] = pix

    sels = []
    for b in range(batch):
        seg = pix[b * hw:(b + 1) * hw, :]
        m = jnp.max(seg)
        io = jax.lax.broadcasted_iota(jnp.int32, (hw, 1), 0)
        first = jnp.min(jnp.where(seg == m, io, hw))
        sels.append(fv_ref[pl.ds(b * hw + first, 1), :])
    sel = jnp.concatenate(sels, axis=0)
    selm2 = sel * (-2.0)
    seln = jnp.sum(sel * sel, axis=1, keepdims=True)

    def d2_step(i, _):
        mbc = mb_ref[pl.ds(i * _CHUNK, _CHUNK), :]
        mbn = jax.lax.dot_general(ones_row, mbc * mbc,
                                  (((1,), (1,)), ((), ())),
                                  preferred_element_type=jnp.float32)
        prod = jax.lax.dot_general(selm2, mbc,
                                   (((1,), (1,)), ((), ())),
                                   preferred_element_type=jnp.float32)
        d2_ref[:, pl.ds(i * _CHUNK, _CHUNK)] = prod + mbn + seln
        return 0

    jax.lax.fori_loop(0, n_chunks, d2_step, 0)

    io2 = jax.lax.broadcasted_iota(jnp.int32, (batch, nb), 1)
    vals = []
    for _ in range(b_neigh):
        d = d2_ref[...]
        mk = jnp.min(d, axis=1, keepdims=True)
        vals.append(mk)
        fk = jnp.min(jnp.where(d == mk, io2, jnp.int32(2 ** 30)),
                     axis=1, keepdims=True)
        d2_ref[...] = jnp.where(io2 == fk, jnp.inf, d)

    sd = jnp.sqrt(jnp.concatenate(vals, axis=1))
    mx = jnp.max(sd, axis=1, keepdims=True)
    e = jnp.exp(sd - mx)
    p0 = e[:, 0:1] / jnp.sum(e, axis=1, keepdims=True)
    img_ref[...] = sd[:, 0:1] * (1.0 - p0)


def kernel(feature_batch, mb):
    batch, height, width, channels = feature_batch.shape
    nq = batch * height * width
    hw = height * width
    nb = mb.shape[0]
    b_neigh = 9
    fv = jnp.reshape(feature_batch, (nq, channels))

    body = functools.partial(_scorer_body, nq, hw, nb, b_neigh)
    pix, img = pl.pallas_call(
        body,
        out_shape=(
            jax.ShapeDtypeStruct((nq, 1), jnp.float32),
            jax.ShapeDtypeStruct((batch, 1), jnp.float32),
        ),
        scratch_shapes=[
            pltpu.VMEM((nq, 1), jnp.float32),
            pltpu.VMEM((batch, nb), jnp.float32),
        ],
    )(fv, mb)
    return (jnp.reshape(pix, (batch, 1, height, width)),
            jnp.reshape(img, (batch,)))


# speedup vs baseline: 1.0014x; 1.0014x over previous
import functools

import jax
import jax.numpy as jnp
from jax.experimental import pallas as pl
from jax.experimental.pallas import tpu as pltpu

_CHUNK = 512


def _scorer_body(nq, hw, nb, b_neigh, fv_ref, mb_ref, pix_ref, img_ref,
                 fva_ref, mba_ref, acc_ref, d2_ref):
    batch = nq // hw
    d = fv_ref.shape[1]
    n_chunks = mb_ref.shape[0] // _CHUNK
    fv = fv_ref[...]

    fva_ref[:, :d] = (fv * (-2.0)).astype(jnp.bfloat16)
    fva_ref[:, d:] = jnp.concatenate(
        [jnp.ones((nq, 2), jnp.float32),
         jnp.zeros((nq, 6), jnp.float32)], axis=1).astype(jnp.bfloat16)

    mbv = mb_ref[...]
    mba_ref[:, :d] = mbv.astype(jnp.bfloat16)
    mbn = jnp.sum(mbv * mbv, axis=1, keepdims=True)
    hi = mbn.astype(jnp.bfloat16)
    lo = (mbn - hi.astype(jnp.float32)).astype(jnp.bfloat16)
    mba_ref[:, d:] = jnp.concatenate(
        [hi, lo, jnp.zeros((nb, 6), jnp.bfloat16)], axis=1)

    acc_ref[...] = jnp.full(acc_ref.shape, jnp.inf, jnp.float32)
    fva = fva_ref[...]

    def min_step(i, _):
        pa = jax.lax.dot_general(fva, mba_ref[pl.ds(i * _CHUNK, _CHUNK), :],
                                 (((1,), (1,)), ((), ())),
                                 preferred_element_type=jnp.float32)
        t = pa[:, 0:d]
        for j in range(1, _CHUNK // d):
            t = jnp.minimum(t, pa[:, j * d:(j + 1) * d])
        acc_ref[...] = jnp.minimum(acc_ref[...], t)
        return 0

    jax.lax.fori_loop(0, n_chunks, min_step, 0)

    fvn = jnp.sum(fv * fv, axis=1, keepdims=True)
    rowmin = jnp.min(acc_ref[...], axis=1, keepdims=True)
    pix = jnp.sqrt(rowmin + fvn)
    pix_ref[...] = pix

    sels = []
    for b in range(batch):
        seg = pix[b * hw:(b + 1) * hw, :]
        m = jnp.max(seg)
        io = jax.lax.broadcasted_iota(jnp.int32, (hw, 1), 0)
        first = jnp.min(jnp.where(seg == m, io, hw))
        sels.append(fv_ref[pl.ds(b * hw + first, 1), :])
    sel = jnp.concatenate(sels, axis=0)
    selm2 = sel * (-2.0)
    seln = jnp.sum(sel * sel, axis=1, keepdims=True)
    ones_row = jnp.ones((1, d), jnp.float32)

    def d2_step(i, _):
        mbc = mb_ref[pl.ds(i * _CHUNK, _CHUNK), :]
        mbn = jax.lax.dot_general(ones_row, mbc * mbc,
                                  (((1,), (1,)), ((), ())),
                                  preferred_element_type=jnp.float32)
        prod = jax.lax.dot_general(selm2, mbc,
                                   (((1,), (1,)), ((), ())),
                                   preferred_element_type=jnp.float32)
        d2_ref[:, pl.ds(i * _CHUNK, _CHUNK)] = prod + mbn + seln
        return 0

    jax.lax.fori_loop(0, n_chunks, d2_step, 0)

    io2 = jax.lax.broadcasted_iota(jnp.int32, (batch, nb), 1)
    vals = []
    for _ in range(b_neigh):
        d = d2_ref[...]
        mk = jnp.min(d, axis=1, keepdims=True)
        vals.append(mk)
        fk = jnp.min(jnp.where(d == mk, io2, jnp.int32(2 ** 30)),
                     axis=1, keepdims=True)
        d2_ref[...] = jnp.where(io2 == fk, jnp.inf, d)

    sd = jnp.sqrt(jnp.concatenate(vals, axis=1))
    mx = jnp.max(sd, axis=1, keepdims=True)
    e = jnp.exp(sd - mx)
    p0 = e[:, 0:1] / jnp.sum(e, axis=1, keepdims=True)
    img_ref[...] = sd[:, 0:1] * (1.0 - p0)


def kernel(feature_batch, mb):
    batch, height, width, channels = feature_batch.shape
    nq = batch * height * width
    hw = height * width
    nb = mb.shape[0]
    b_neigh = 9
    fv = jnp.reshape(feature_batch, (nq, channels))

    body = functools.partial(_scorer_body, nq, hw, nb, b_neigh)
    pix, img = pl.pallas_call(
        body,
        out_shape=(
            jax.ShapeDtypeStruct((nq, 1), jnp.float32),
            jax.ShapeDtypeStruct((batch, 1), jnp.float32),
        ),
        scratch_shapes=[
            pltpu.VMEM((nq, channels + 8), jnp.bfloat16),
            pltpu.VMEM((nb, channels + 8), jnp.bfloat16),
            pltpu.VMEM((nq, channels), jnp.float32),
            pltpu.VMEM((batch, nb), jnp.float32),
        ],
    )(fv, mb)
    return (jnp.reshape(pix, (batch, 1, height, width)),
            jnp.reshape(img, (batch,)))


# speedup vs baseline: 1.3397x; 1.3378x over previous
import functools

import jax
import jax.numpy as jnp
from jax.experimental import pallas as pl
from jax.experimental.pallas import tpu as pltpu

_CHUNK = 512


def _scorer_body(nq, hw, nb, b_neigh, fv_ref, mb_ref, pix_ref, img_ref,
                 fvb_ref, mbb_ref, mbn_ref, acc_ref, d2_ref):
    batch = nq // hw
    d = fv_ref.shape[1]
    fv = fv_ref[...]
    ones_row = jnp.ones((1, d), jnp.float32)

    fvb_ref[...] = (fv * (-2.0)).astype(jnp.bfloat16)
    mbv = mb_ref[...]
    mbb_ref[...] = mbv.astype(jnp.bfloat16)
    mbn_ref[...] = jax.lax.dot_general(ones_row, mbv * mbv,
                                       (((1,), (1,)), ((), ())),
                                       preferred_element_type=jnp.float32)

    acc_ref[...] = jnp.full(acc_ref.shape, jnp.inf, jnp.float32)
    fvb = fvb_ref[...]
    _UNROLL = 4
    n_outer = nb // (_CHUNK * _UNROLL)

    def min_step(i, _):
        base = i * (_CHUNK * _UNROLL)
        for u in range(_UNROLL):
            off = base + u * _CHUNK
            pa = jax.lax.dot_general(fvb, mbb_ref[pl.ds(off, _CHUNK), :],
                                     (((1,), (1,)), ((), ())),
                                     preferred_element_type=jnp.float32)
            t = None
            for j in range(_CHUNK // d):
                nrow = mbn_ref[0:1, pl.ds(off + j * d, d)]
                blk = pa[:, j * d:(j + 1) * d] + nrow
                t = blk if t is None else jnp.minimum(t, blk)
            acc_ref[...] = jnp.minimum(acc_ref[...], t)
        return 0

    jax.lax.fori_loop(0, n_outer, min_step, 0)

    fvn = jnp.sum(fv * fv, axis=1, keepdims=True)
    rowmin = jnp.min(acc_ref[...], axis=1, keepdims=True)
    pix = jnp.sqrt(rowmin + fvn)
    pix_ref[...] = pix

    sels = []
    for b in range(batch):
        seg = pix[b * hw:(b + 1) * hw, :]
        m = jnp.max(seg)
        io = jax.lax.broadcasted_iota(jnp.int32, (hw, 1), 0)
        first = jnp.min(jnp.where(seg == m, io, hw))
        sels.append(fv_ref[pl.ds(b * hw + first, 1), :])
    sel = jnp.concatenate(sels, axis=0)
    selm2 = sel * (-2.0)
    seln = jnp.sum(sel * sel, axis=1, keepdims=True)

    prod2 = jax.lax.dot_general(selm2, mbv,
                                (((1,), (1,)), ((), ())),
                                preferred_element_type=jnp.float32)
    d2_ref[...] = prod2 + mbn_ref[...] + seln

    io2 = jax.lax.broadcasted_iota(jnp.int32, (batch, nb), 1)
    vals = []
    for _ in range(b_neigh):
        d = d2_ref[...]
        mk = jnp.min(d, axis=1, keepdims=True)
        vals.append(mk)
        fk = jnp.min(jnp.where(d == mk, io2, jnp.int32(2 ** 30)),
                     axis=1, keepdims=True)
        d2_ref[...] = jnp.where(io2 == fk, jnp.inf, d)

    sd = jnp.sqrt(jnp.concatenate(vals, axis=1))
    mx = jnp.max(sd, axis=1, keepdims=True)
    e = jnp.exp(sd - mx)
    p0 = e[:, 0:1] / jnp.sum(e, axis=1, keepdims=True)
    img_ref[...] = sd[:, 0:1] * (1.0 - p0)


def kernel(feature_batch, mb):
    batch, height, width, channels = feature_batch.shape
    nq = batch * height * width
    hw = height * width
    nb = mb.shape[0]
    b_neigh = 9
    fv = jnp.reshape(feature_batch, (nq, channels))

    body = functools.partial(_scorer_body, nq, hw, nb, b_neigh)
    pix, img = pl.pallas_call(
        body,
        out_shape=(
            jax.ShapeDtypeStruct((nq, 1), jnp.float32),
            jax.ShapeDtypeStruct((batch, 1), jnp.float32),
        ),
        scratch_shapes=[
            pltpu.VMEM((nq, channels), jnp.bfloat16),
            pltpu.VMEM((nb, channels), jnp.bfloat16),
            pltpu.VMEM((1, nb), jnp.float32),
            pltpu.VMEM((nq, channels), jnp.float32),
            pltpu.VMEM((batch, nb), jnp.float32),
        ],
    )(fv, mb)
    return (jnp.reshape(pix, (batch, 1, height, width)),
            jnp.reshape(img, (batch,)))
